# baseline (speedup 1.0000x reference)
import functools

import jax
import jax.numpy as jnp
from jax import lax
from jax.experimental import pallas as pl
from jax.experimental.pallas import tpu as pltpu
from jax.experimental.pallas import tpu_sc as plsc

B, C, N, K = 2, 128, 10000, 32
OUT = 128
GROUPS = 4
NT = B * N

_NC, _NS = 2, 16
_CHUNK = 4
_NPT = 640
_NCHUNKS = _NPT // _CHUNK
_IBLK = 16
_NIB = _NCHUNKS // _IBLK
_NPAD = _NS * _NPT
_BSTRIDE = 12000
_NTPAD = B * _BSTRIDE
_NROWS = 10240
_STAGE = 128

_RB = 2000
_NBLK = NT // _RB
_NPB = N // _RB


def _sc_gather_max(xt, src, dst):
    mesh = plsc.VectorSubcoreMesh(core_axis_name="c", subcore_axis_name="s")
    ipt = _NPT * K
    ipb = _IBLK * _CHUNK * K

    @functools.partial(
        pl.kernel,
        mesh=mesh,
        out_type=jax.ShapeDtypeStruct((_NTPAD, C), jnp.float32),
        scratch_types=[
            pltpu.VMEM_SHARED((_NROWS, C), jnp.float32),
            pltpu.VMEM((2, ipb), jnp.int32),
            pltpu.VMEM((2, ipb), jnp.int32),
            pltpu.VMEM((_CHUNK * K, C), jnp.float32),
            pltpu.VMEM((_CHUNK * K, C), jnp.float32),
            pltpu.VMEM((2, _CHUNK, C), jnp.float32),
            pltpu.SemaphoreType.DMA,
            pltpu.SemaphoreType.DMA,
            pltpu.SemaphoreType.DMA,
            pltpu.SemaphoreType.DMA,
            pltpu.SemaphoreType.DMA,
            pltpu.SemaphoreType.DMA,
            pltpu.SemaphoreType.DMA,
            pltpu.SemaphoreType.DMA,
        ],
    )
    def body(xt_hbm, src_hbm, dst_hbm, out_hbm,
             table, sidx, didx, srows, drows, obuf,
             sem_i0, sem_i1, sem_j0, sem_j1,
             sem_gs, sem_gd, sem_o0, sem_o1):
        c = lax.axis_index("c")
        s = lax.axis_index("s")
        sem_i = (sem_i0, sem_i1)
        sem_j = (sem_j0, sem_j1)
        sem_o = (sem_o0, sem_o1)
        idx0 = c * (_NPAD * K) + s * ipt
        out0 = c * _BSTRIDE + s * _NPT

        for j in range(_NROWS // _NS // _STAGE):
            row = (s * (_NROWS // _NS // _STAGE) + j) * _STAGE
            pltpu.sync_copy(xt_hbm.at[pl.ds(c * _NROWS + row, _STAGE)],
                            srows)
            pltpu.sync_copy(srows, table.at[pl.ds(row, _STAGE)])
        plsc.subcore_barrier()

        def fire_idx(blk, ib):
            pltpu.async_copy(src_hbm.at[pl.ds(idx0 + blk * ipb, ipb)],
                             sidx.at[ib], sem_i[ib])
            pltpu.async_copy(dst_hbm.at[pl.ds(idx0 + blk * ipb, ipb)],
                             didx.at[ib], sem_j[ib])

        def wait_idx(ib):
            pltpu.make_async_copy(src_hbm.at[pl.ds(idx0, ipb)],
                                  sidx.at[ib], sem_i[ib]).wait()
            pltpu.make_async_copy(dst_hbm.at[pl.ds(idx0, ipb)],
                                  didx.at[ib], sem_j[ib]).wait()

        def wait_out(ob):
            pltpu.make_async_copy(obuf.at[ob],
                                  out_hbm.at[pl.ds(out0, _CHUNK)],
                                  sem_o[ob]).wait()

        fire_idx(0, 0)

        def blk_pair(p, carry):
            for ib in range(2):
                blk = p * 2 + ib
                wait_idx(ib)

                @pl.when(blk + 1 < _NIB)
                def _prefetch():
                    fire_idx(blk + 1, 1 - ib)

                def chunk_pair(q, carry2):
                    for ob in range(2):
                        ch = q * 2 + ob
                        g = blk * _IBLK + ch
                        hk = _CHUNK * K // 2
                        cps = []
                        for h in range(2):
                            cps.append(pltpu.async_copy(
                                table.at[sidx.at[
                                    ib, pl.ds(ch * (_CHUNK * K) + h * hk,
                                              hk)]],
                                srows.at[pl.ds(h * hk, hk)], sem_gs))
                            cps.append(pltpu.async_copy(
                                table.at[didx.at[
                                    ib, pl.ds(ch * (_CHUNK * K) + h * hk,
                                              hk)]],
                                drows.at[pl.ds(h * hk, hk)], sem_gd))
                        for cp in cps:
                            cp.wait()

                        @pl.when(g >= 2)
                        def _drain():
                            wait_out(ob)

                        for nd in range(_CHUNK):
                            def kbody(k, accs):
                                r = nd * K + k
                                return tuple(
                                    jnp.maximum(
                                        accs[cc],
                                        srows[r, pl.ds(cc * 16, 16)]
                                        - drows[r, pl.ds(cc * 16, 16)])
                                    for cc in range(8))
                            accs = tuple(
                                jnp.full((16,), -jnp.inf, jnp.float32)
                                for _ in range(8))
                            accs = lax.fori_loop(0, K, kbody, accs, unroll=4)
                            for cc in range(8):
                                obuf[ob, nd, pl.ds(cc * 16, 16)] = accs[cc]

                        pltpu.async_copy(
                            obuf.at[ob],
                            out_hbm.at[pl.ds(out0 + g * _CHUNK, _CHUNK)],
                            sem_o[ob])
                    return carry2

                lax.fori_loop(0, _IBLK // 2, chunk_pair, 0)
            return carry

        lax.fori_loop(0, _NIB // 2, blk_pair, 0)
        wait_out(0)
        wait_out(1)

    return body(xt, src, dst)


def _conv_stats_body(xt_b, h2_b, w1, w2, bias, y_b, ssum, ssq):
    i = pl.program_id(0)
    y = (jnp.dot(xt_b[...], w1[...], preferred_element_type=jnp.float32)
         + jnp.dot(h2_b[...], w2[...], preferred_element_type=jnp.float32)
         + bias[...])
    y_b[...] = y

    @pl.when(i == 0)
    def _init():
        ssum[...] = jnp.zeros_like(ssum)
        ssq[...] = jnp.zeros_like(ssq)

    ssum[0, :] += jnp.sum(y, axis=0)
    ssq[0, :] += jnp.sum(y * y, axis=0)


def _bn_gelu_body(y_b, ssum, ssq, gamma, beta, out_b):
    mean = ssum[0, :] * (1.0 / NT)
    var = ssq[0, :] * (1.0 / NT) - mean * mean
    rstd = lax.rsqrt(var + 1e-5)
    g = gamma[0, :] * rstd
    bt = beta[0, :] - mean * g
    yn = y_b[...] * g[None, :] + bt[None, :]
    o = 0.5 * yn * (1.0 + lax.erf(yn * 0.7071067811865476))
    out_b[...] = o.T[None]


def kernel(x, edge_index, conv_w, conv_b, gamma, beta):
    xt = jnp.transpose(x[..., 0], (0, 2, 1)).reshape(NT, C)

    src = jnp.pad(edge_index[0], ((0, 0), (0, _NPAD - N), (0, 0))
                  ).reshape(B * _NPAD * K)
    dst = jnp.pad(edge_index[1], ((0, 0), (0, _NPAD - N), (0, 0))
                  ).reshape(B * _NPAD * K)

    xtp = jnp.pad(xt.reshape(B, N, C), ((0, 0), (0, _NROWS - N), (0, 0))
                  ).reshape(B * _NROWS, C)
    h2 = _sc_gather_max(xtp, src, dst)

    cin_g = 2 * C // GROUPS
    wg = conv_w[:, :, 0, 0].reshape(GROUPS, OUT // GROUPS, cin_g)
    wbd = jnp.zeros((2 * C, OUT), jnp.float32)
    for g in range(GROUPS):
        wbd = wbd.at[g * cin_g:(g + 1) * cin_g,
                     g * (OUT // GROUPS):(g + 1) * (OUT // GROUPS)].set(
                         jnp.transpose(wg[g]))
    w1, w2 = wbd[:C], wbd[C:]

    nb_pb = _BSTRIDE // _RB
    y, ssum, ssq = pl.pallas_call(
        _conv_stats_body,
        grid=(_NBLK,),
        in_specs=[
            pl.BlockSpec((_RB, C), lambda i: (i, 0)),
            pl.BlockSpec((_RB, C),
                         lambda i: ((i // _NPB) * nb_pb + (i % _NPB), 0)),
            pl.BlockSpec((C, OUT), lambda i: (0, 0)),
            pl.BlockSpec((C, OUT), lambda i: (0, 0)),
            pl.BlockSpec((1, OUT), lambda i: (0, 0)),
        ],
        out_specs=[
            pl.BlockSpec((_RB, OUT), lambda i: (i, 0)),
            pl.BlockSpec((8, OUT), lambda i: (0, 0)),
            pl.BlockSpec((8, OUT), lambda i: (0, 0)),
        ],
        out_shape=[
            jax.ShapeDtypeStruct((NT, OUT), jnp.float32),
            jax.ShapeDtypeStruct((8, OUT), jnp.float32),
            jax.ShapeDtypeStruct((8, OUT), jnp.float32),
        ],
    )(xt, h2, w1, w2, conv_b.reshape(1, OUT))

    out3 = pl.pallas_call(
        _bn_gelu_body,
        grid=(B,),
        in_specs=[
            pl.BlockSpec((N, OUT), lambda i: (i, 0)),
            pl.BlockSpec((8, OUT), lambda i: (0, 0)),
            pl.BlockSpec((8, OUT), lambda i: (0, 0)),
            pl.BlockSpec((1, OUT), lambda i: (0, 0)),
            pl.BlockSpec((1, OUT), lambda i: (0, 0)),
        ],
        out_specs=pl.BlockSpec((1, OUT, N), lambda i: (i, 0, 0)),
        out_shape=jax.ShapeDtypeStruct((B, OUT, N), jnp.float32),
    )(y, ssum, ssq, gamma.reshape(1, OUT), beta.reshape(1, OUT))

    return out3[..., None]


# speedup vs baseline: 1.4810x; 1.4810x over previous
import functools

import jax
import jax.numpy as jnp
from jax import lax
from jax.experimental import pallas as pl
from jax.experimental.pallas import tpu as pltpu
from jax.experimental.pallas import tpu_sc as plsc

B, C, N, K = 2, 128, 10000, 32
OUT = 128
GROUPS = 4
NT = B * N

_NC, _NS = 2, 16
_CHUNK = 2
_NPT = 640
_NCHUNKS = _NPT // _CHUNK
_IBLK = 32
_NIB = _NCHUNKS // _IBLK
_NPAD = _NS * _NPT
_BSTRIDE = 12000
_NTPAD = B * _BSTRIDE
_NROWS = 10240
_STAGE = 64

_RB = 2000
_NBLK = NT // _RB
_NPB = N // _RB


def _sc_gather_max(xt, src, dst):
    mesh = plsc.VectorSubcoreMesh(core_axis_name="c", subcore_axis_name="s")
    ipt = _NPT * K
    ipb = _IBLK * _CHUNK * K

    @functools.partial(
        pl.kernel,
        mesh=mesh,
        out_type=jax.ShapeDtypeStruct((_NTPAD, C), jnp.float32),
        scratch_types=[
            pltpu.VMEM_SHARED((_NROWS, C), jnp.float32),
            pltpu.VMEM((2, ipb), jnp.int32),
            pltpu.VMEM((2, ipb), jnp.int32),
            pltpu.VMEM((2, _CHUNK * K, C), jnp.float32),
            pltpu.VMEM((2, _CHUNK * K, C), jnp.float32),
            pltpu.VMEM((2, _CHUNK, C), jnp.float32),
            pltpu.SemaphoreType.DMA,
            pltpu.SemaphoreType.DMA,
            pltpu.SemaphoreType.DMA,
            pltpu.SemaphoreType.DMA,
            pltpu.SemaphoreType.DMA,
            pltpu.SemaphoreType.DMA,
            pltpu.SemaphoreType.DMA,
            pltpu.SemaphoreType.DMA,
            pltpu.SemaphoreType.DMA,
            pltpu.SemaphoreType.DMA,
        ],
    )
    def body(xt_hbm, src_hbm, dst_hbm, out_hbm,
             table, sidx, didx, srows, drows, obuf,
             sem_i0, sem_i1, sem_j0, sem_j1,
             sem_gs0, sem_gs1, sem_gd0, sem_gd1, sem_o0, sem_o1):
        c = lax.axis_index("c")
        s = lax.axis_index("s")
        sem_i = (sem_i0, sem_i1)
        sem_j = (sem_j0, sem_j1)
        sem_gs = (sem_gs0, sem_gs1)
        sem_gd = (sem_gd0, sem_gd1)
        sem_o = (sem_o0, sem_o1)
        idx0 = c * (_NPAD * K) + s * ipt
        out0 = c * _BSTRIDE + s * _NPT

        for j in range(_NROWS // _NS // _STAGE):
            row = (s * (_NROWS // _NS // _STAGE) + j) * _STAGE
            pltpu.sync_copy(xt_hbm.at[pl.ds(c * _NROWS + row, _STAGE)],
                            srows.at[0])
            pltpu.sync_copy(srows.at[0], table.at[pl.ds(row, _STAGE)])
        plsc.subcore_barrier()

        def fire_idx(blk, ib):
            pltpu.async_copy(src_hbm.at[pl.ds(idx0 + blk * ipb, ipb)],
                             sidx.at[ib], sem_i[ib])
            pltpu.async_copy(dst_hbm.at[pl.ds(idx0 + blk * ipb, ipb)],
                             didx.at[ib], sem_j[ib])

        def wait_idx(ib):
            pltpu.make_async_copy(src_hbm.at[pl.ds(idx0, ipb)],
                                  sidx.at[ib], sem_i[ib]).wait()
            pltpu.make_async_copy(dst_hbm.at[pl.ds(idx0, ipb)],
                                  didx.at[ib], sem_j[ib]).wait()

        def wait_out(ob):
            pltpu.make_async_copy(obuf.at[ob],
                                  out_hbm.at[pl.ds(out0, _CHUNK)],
                                  sem_o[ob]).wait()

        fire_idx(0, 0)

        def blk_pair(p, carry):
            for ib in range(2):
                blk = p * 2 + ib
                wait_idx(ib)

                @pl.when(blk + 1 < _NIB)
                def _prefetch():
                    fire_idx(blk + 1, 1 - ib)

                def fire_rows(ch, rb):
                    pltpu.async_copy(
                        table.at[sidx.at[ib, pl.ds(ch * (_CHUNK * K),
                                                   _CHUNK * K)]],
                        srows.at[rb], sem_gs[rb])
                    pltpu.async_copy(
                        table.at[didx.at[ib, pl.ds(ch * (_CHUNK * K),
                                                   _CHUNK * K)]],
                        drows.at[rb], sem_gd[rb])

                def wait_rows(rb):
                    pltpu.make_async_copy(
                        table.at[sidx.at[ib, pl.ds(0, _CHUNK * K)]],
                        srows.at[rb], sem_gs[rb]).wait()
                    pltpu.make_async_copy(
                        table.at[didx.at[ib, pl.ds(0, _CHUNK * K)]],
                        drows.at[rb], sem_gd[rb]).wait()

                fire_rows(0, 0)

                def chunk_pair(q, carry2):
                    for rb in range(2):
                        ch = q * 2 + rb
                        g = blk * _IBLK + ch

                        @pl.when(ch + 1 < _IBLK)
                        def _pref_rows():
                            fire_rows(ch + 1, 1 - rb)

                        wait_rows(rb)

                        @pl.when(g >= 2)
                        def _drain():
                            wait_out(rb)

                        for nd in range(_CHUNK):
                            def kbody(k, accs):
                                r = nd * K + k
                                return tuple(
                                    jnp.maximum(
                                        accs[cc],
                                        srows[rb, r, pl.ds(cc * 16, 16)]
                                        - drows[rb, r, pl.ds(cc * 16, 16)])
                                    for cc in range(8))
                            accs = tuple(
                                jnp.full((16,), -jnp.inf, jnp.float32)
                                for _ in range(8))
                            accs = lax.fori_loop(0, K, kbody, accs, unroll=4)
                            for cc in range(8):
                                obuf[rb, nd, pl.ds(cc * 16, 16)] = accs[cc]

                        pltpu.async_copy(
                            obuf.at[rb],
                            out_hbm.at[pl.ds(out0 + g * _CHUNK, _CHUNK)],
                            sem_o[rb])
                    return carry2

                lax.fori_loop(0, _IBLK // 2, chunk_pair, 0)
            return carry

        lax.fori_loop(0, _NIB // 2, blk_pair, 0)
        wait_out(0)
        wait_out(1)

    return body(xt, src, dst)


def _conv_stats_body(xt_b, h2_b, w1, w2, bias, y_b, ssum, ssq):
    i = pl.program_id(0)
    y = (jnp.dot(xt_b[...], w1[...], preferred_element_type=jnp.float32)
         + jnp.dot(h2_b[...], w2[...], preferred_element_type=jnp.float32)
         + bias[...])
    y_b[...] = y

    @pl.when(i == 0)
    def _init():
        ssum[...] = jnp.zeros_like(ssum)
        ssq[...] = jnp.zeros_like(ssq)

    ssum[0, :] += jnp.sum(y, axis=0)
    ssq[0, :] += jnp.sum(y * y, axis=0)


def _bn_gelu_body(y_b, ssum, ssq, gamma, beta, out_b):
    mean = ssum[0, :] * (1.0 / NT)
    var = ssq[0, :] * (1.0 / NT) - mean * mean
    rstd = lax.rsqrt(var + 1e-5)
    g = gamma[0, :] * rstd
    bt = beta[0, :] - mean * g
    yn = y_b[...] * g[None, :] + bt[None, :]
    o = 0.5 * yn * (1.0 + lax.erf(yn * 0.7071067811865476))
    out_b[...] = o.T[None]


def kernel(x, edge_index, conv_w, conv_b, gamma, beta):
    xt = jnp.transpose(x[..., 0], (0, 2, 1)).reshape(NT, C)

    src = jnp.pad(edge_index[0], ((0, 0), (0, _NPAD - N), (0, 0))
                  ).reshape(B * _NPAD * K)
    dst = jnp.pad(edge_index[1], ((0, 0), (0, _NPAD - N), (0, 0))
                  ).reshape(B * _NPAD * K)

    xtp = jnp.pad(xt.reshape(B, N, C), ((0, 0), (0, _NROWS - N), (0, 0))
                  ).reshape(B * _NROWS, C)
    h2 = _sc_gather_max(xtp, src, dst)

    cin_g = 2 * C // GROUPS
    wg = conv_w[:, :, 0, 0].reshape(GROUPS, OUT // GROUPS, cin_g)
    wbd = jnp.zeros((2 * C, OUT), jnp.float32)
    for g in range(GROUPS):
        wbd = wbd.at[g * cin_g:(g + 1) * cin_g,
                     g * (OUT // GROUPS):(g + 1) * (OUT // GROUPS)].set(
                         jnp.transpose(wg[g]))
    w1, w2 = wbd[:C], wbd[C:]

    nb_pb = _BSTRIDE // _RB
    y, ssum, ssq = pl.pallas_call(
        _conv_stats_body,
        grid=(_NBLK,),
        in_specs=[
            pl.BlockSpec((_RB, C), lambda i: (i, 0)),
            pl.BlockSpec((_RB, C),
                         lambda i: ((i // _NPB) * nb_pb + (i % _NPB), 0)),
            pl.BlockSpec((C, OUT), lambda i: (0, 0)),
            pl.BlockSpec((C, OUT), lambda i: (0, 0)),
            pl.BlockSpec((1, OUT), lambda i: (0, 0)),
        ],
        out_specs=[
            pl.BlockSpec((_RB, OUT), lambda i: (i, 0)),
            pl.BlockSpec((8, OUT), lambda i: (0, 0)),
            pl.BlockSpec((8, OUT), lambda i: (0, 0)),
        ],
        out_shape=[
            jax.ShapeDtypeStruct((NT, OUT), jnp.float32),
            jax.ShapeDtypeStruct((8, OUT), jnp.float32),
            jax.ShapeDtypeStruct((8, OUT), jnp.float32),
        ],
    )(xt, h2, w1, w2, conv_b.reshape(1, OUT))

    out3 = pl.pallas_call(
        _bn_gelu_body,
        grid=(B,),
        in_specs=[
            pl.BlockSpec((N, OUT), lambda i: (i, 0)),
            pl.BlockSpec((8, OUT), lambda i: (0, 0)),
            pl.BlockSpec((8, OUT), lambda i: (0, 0)),
            pl.BlockSpec((1, OUT), lambda i: (0, 0)),
            pl.BlockSpec((1, OUT), lambda i: (0, 0)),
        ],
        out_specs=pl.BlockSpec((1, OUT, N), lambda i: (i, 0, 0)),
        out_shape=jax.ShapeDtypeStruct((B, OUT, N), jnp.float32),
    )(y, ssum, ssq, gamma.reshape(1, OUT), beta.reshape(1, OUT))

    return out3[..., None]
